# Initial kernel scaffold; baseline (speedup 1.0000x reference)
import functools

import jax
import jax.numpy as jnp
from jax import lax
from jax.experimental import pallas as pl
from jax.experimental.pallas import tpu as pltpu
from jax.experimental.pallas import tpu_sc as plsc


@functools.lru_cache(maxsize=None)
def _build(B, S, V, D, N):
    info = plsc.get_sparse_core_info()
    NW = info.num_cores * info.num_subcores

    ROWS = B * S
    assert ROWS % NW == 0
    RPW = ROWS // NW
    C = 32
    while RPW % C or 2 * C * D * 4 > 480 * 1024:
        C //= 2
    NCH = RPW // C
    assert RPW % C == 0 and C <= 128

    for b in range(B):
        owner = (b * S) // RPW
        assert (b * S + N) // RPW == owner and (b * S) % RPW + 1 + N <= RPW

    mesh = plsc.VectorSubcoreMesh(core_axis_name="c", subcore_axis_name="s")

    @functools.partial(
        pl.kernel,
        out_type=jax.ShapeDtypeStruct((ROWS, D), jnp.float32),
        mesh=mesh,
        scratch_types=[
            pltpu.VMEM((RPW,), jnp.int32),
            pltpu.VMEM((C, D), jnp.float32),
            pltpu.VMEM((C, D), jnp.float32),
            pltpu.SemaphoreType.DMA,
            pltpu.SemaphoreType.DMA,
            pltpu.SemaphoreType.DMA,
            pltpu.SemaphoreType.DMA,
        ],
    )
    def k(tok_hbm, wte_hbm, le_hbm, out_hbm, idx_v, buf0, buf1, g0, g1, w0, w1):
        wid = lax.axis_index("s") * info.num_cores + lax.axis_index("c")
        base = wid * RPW
        bufs = (buf0, buf1)
        gsem = (g0, g1)
        wsem = (w0, w1)

        pltpu.sync_copy(tok_hbm.at[pl.ds(base, RPW)], idx_v)

        def gather(c):
            return pltpu.async_copy(
                wte_hbm.at[idx_v.at[pl.ds(c * C, C)]], bufs[c % 2], gsem[c % 2]
            )

        def write(c):
            return pltpu.async_copy(
                bufs[c % 2], out_hbm.at[pl.ds(base + c * C, C)], wsem[c % 2]
            )

        writes = [None] * NCH
        gh = gather(0)
        for c in range(1, NCH):
            if c >= 2:
                writes[c - 2].wait()
            gh_next = gather(c)
            gh.wait()
            writes[c - 1] = write(c - 1)
            gh = gh_next
        gh.wait()
        writes[NCH - 1] = write(NCH - 1)
        if NCH >= 2:
            writes[NCH - 2].wait()
        writes[NCH - 1].wait()

        for b in range(B):
            owner = (b * S) // RPW

            @pl.when(wid == owner)
            def _():
                pltpu.sync_copy(le_hbm.at[pl.ds(b * N, N)], buf0.at[pl.ds(0, N)])
                pltpu.sync_copy(buf0.at[pl.ds(0, N)], out_hbm.at[pl.ds(b * S + 1, N)])

    return k


def kernel(tokens, wte, learned_embedding):
    B, S = tokens.shape
    V, D = wte.shape
    N = learned_embedding.shape[1]
    k = _build(B, S, V, D, N)
    out = k(
        tokens.reshape(B * S),
        wte,
        learned_embedding.reshape(B * N, D),
    )
    return out.reshape(B, S, D)


# speedup vs baseline: 5.8673x; 5.8673x over previous
import functools

import jax
import jax.numpy as jnp
from jax import lax
from jax.experimental import pallas as pl
from jax.experimental.pallas import tpu as pltpu
from jax.experimental.pallas import tpu_sc as plsc


@functools.lru_cache(maxsize=None)
def _build(B, S, V, D, N):
    info = plsc.get_sparse_core_info()
    NW = info.num_cores * info.num_subcores

    ROWS = B * S
    assert ROWS % NW == 0
    RPW = ROWS // NW
    C = 32
    while RPW % C or 2 * C * D * 4 > 480 * 1024:
        C //= 2
    NCH = RPW // C
    assert RPW % C == 0 and C <= 128
    assert N <= 32 <= C

    for b in range(B):
        owner = (b * S) // RPW
        assert (b * S + N) // RPW == owner and (b * S) % RPW + 1 + N <= RPW

    mesh = plsc.VectorSubcoreMesh(core_axis_name="c", subcore_axis_name="s")

    @functools.partial(
        pl.kernel,
        out_type=jax.ShapeDtypeStruct((ROWS, D), jnp.float32),
        mesh=mesh,
        scratch_types=[
            pltpu.VMEM((RPW,), jnp.int32),
            pltpu.VMEM((C, D), jnp.float32),
            pltpu.VMEM((C, D), jnp.float32),
            pltpu.SemaphoreType.DMA,
            pltpu.SemaphoreType.DMA,
            pltpu.SemaphoreType.DMA,
            pltpu.SemaphoreType.DMA,
        ],
    )
    def k(tok_hbm, wte_hbm, le_hbm, out_hbm, idx_v, buf0, buf1, g0, g1, w0, w1):
        wid = lax.axis_index("s") * info.num_cores + lax.axis_index("c")
        base = wid * RPW
        bufs = (buf0, buf1)
        gsem = (g0, g1)
        wsem = (w0, w1)

        pltpu.sync_copy(tok_hbm.at[pl.ds(base, RPW)], idx_v)

        def gather(c):
            return pltpu.async_copy(
                wte_hbm.at[idx_v.at[pl.ds(c * C, C)]], bufs[c % 2], gsem[c % 2]
            )

        def write(c):
            return pltpu.async_copy(
                bufs[c % 2], out_hbm.at[pl.ds(base + c * C, C)], wsem[c % 2]
            )

        writes = [None] * NCH
        gh = gather(0)
        for c in range(1, NCH):
            if c >= 2:
                writes[c - 2].wait()
            gh_next = gather(c)
            gh.wait()
            writes[c - 1] = write(c - 1)
            gh = gh_next
        gh.wait()
        writes[NCH - 1] = write(NCH - 1)
        if NCH >= 2:
            writes[NCH - 2].wait()
        writes[NCH - 1].wait()

        j16 = lax.iota(jnp.int32, 16)
        for b in range(B):
            owner = (b * S) // RPW

            @pl.when(wid == owner)
            def _():
                handles = []
                for h in range(2):
                    gidx = b * N + jnp.minimum(j16 + 16 * h, N - 1)
                    handles.append(
                        pltpu.async_copy(
                            le_hbm.at[gidx], buf0.at[pl.ds(16 * h, 16)], gsem[h]
                        )
                    )
                for h in handles:
                    h.wait()
                handles = []
                for h in range(2):
                    sidx = b * S + 1 + jnp.minimum(j16 + 16 * h, N - 1)
                    handles.append(
                        pltpu.async_copy(
                            buf0.at[pl.ds(16 * h, 16)], out_hbm.at[sidx], wsem[h]
                        )
                    )
                for h in handles:
                    h.wait()

    return k


def kernel(tokens, wte, learned_embedding):
    B, S = tokens.shape
    V, D = wte.shape
    N = learned_embedding.shape[1]
    k = _build(B, S, V, D, N)
    out = k(
        tokens.reshape(B * S),
        wte,
        learned_embedding.reshape(B * N, D),
    )
    return out.reshape(B, S, D)


# speedup vs baseline: 6.1609x; 1.0500x over previous
import functools

import jax
import jax.numpy as jnp
from jax import lax
from jax.experimental import pallas as pl
from jax.experimental.pallas import tpu as pltpu
from jax.experimental.pallas import tpu_sc as plsc


@functools.lru_cache(maxsize=None)
def _build(B, S, V, D, N):
    info = plsc.get_sparse_core_info()
    NW = info.num_cores * info.num_subcores

    ROWS = B * S
    assert ROWS % NW == 0
    RPW = ROWS // NW
    C = 32
    while RPW % C or 2 * C * D * 4 > 480 * 1024:
        C //= 2
    NCH = RPW // C
    assert RPW % C == 0 and C <= 128
    assert N <= 32 <= C

    for b in range(B):
        owner = (b * S) // RPW
        assert (b * S + N) // RPW == owner and (b * S) % RPW + 1 + N <= RPW

    mesh = plsc.VectorSubcoreMesh(core_axis_name="c", subcore_axis_name="s")

    @functools.partial(
        pl.kernel,
        out_type=jax.ShapeDtypeStruct((ROWS, D), jnp.float32),
        mesh=mesh,
        scratch_types=[
            pltpu.VMEM((RPW,), jnp.int32),
            pltpu.VMEM((C, D), jnp.float32),
            pltpu.VMEM((C, D), jnp.float32),
            pltpu.VMEM((32, D), jnp.float32),
            pltpu.SemaphoreType.DMA,
            pltpu.SemaphoreType.DMA,
            pltpu.SemaphoreType.DMA,
            pltpu.SemaphoreType.DMA,
            pltpu.SemaphoreType.DMA,
            pltpu.SemaphoreType.DMA,
            pltpu.SemaphoreType.DMA,
            pltpu.SemaphoreType.DMA,
        ],
    )
    def k(tok_hbm, wte_hbm, le_hbm, out_hbm, idx_v, buf0, buf1, le_v,
          g0, g1, w0, w1, l0, l1, s0, s1):
        wid = lax.axis_index("c") * info.num_subcores + lax.axis_index("s")
        base = wid * RPW
        bufs = (buf0, buf1)
        gsem = (g0, g1)
        wsem = (w0, w1)
        lsem = (l0, l1)
        ssem = (s0, s1)

        j16 = lax.iota(jnp.int32, 16)

        def le_gather(b, h):
            gidx = b * N + jnp.minimum(j16 + 16 * h, N - 1)
            return pltpu.make_async_copy(
                le_hbm.at[gidx], le_v.at[pl.ds(16 * h, 16)], lsem[h]
            )

        def le_scatter(b, h):
            sidx = b * S + 1 + jnp.minimum(j16 + 16 * h, N - 1)
            return pltpu.make_async_copy(
                le_v.at[pl.ds(16 * h, 16)], out_hbm.at[sidx], ssem[h]
            )

        def for_owner(fn):
            for b in range(B):
                owner = (b * S) // RPW

                @pl.when(wid == owner)
                def _():
                    fn(b)

        for_owner(lambda b: [le_gather(b, h).start() for h in range(2)])

        pltpu.sync_copy(tok_hbm.at[pl.ds(base, RPW)], idx_v)

        def gather(c):
            return pltpu.async_copy(
                wte_hbm.at[idx_v.at[pl.ds(c * C, C)]], bufs[c % 2], gsem[c % 2]
            )

        def write(c):
            return pltpu.async_copy(
                bufs[c % 2], out_hbm.at[pl.ds(base + c * C, C)], wsem[c % 2]
            )

        def splice_issue(b):
            for h in range(2):
                le_gather(b, h).wait()
            for h in range(2):
                le_scatter(b, h).start()

        writes = [None] * NCH
        gh = gather(0)
        for c in range(1, NCH):
            if c >= 2:
                writes[c - 2].wait()
            if c == 2:
                for_owner(splice_issue)
            gh_next = gather(c)
            gh.wait()
            writes[c - 1] = write(c - 1)
            gh = gh_next
        gh.wait()
        writes[NCH - 1] = write(NCH - 1)
        if NCH >= 2:
            writes[NCH - 2].wait()
        writes[NCH - 1].wait()
        if NCH <= 2:
            for_owner(splice_issue)
        for_owner(lambda b: [le_scatter(b, h).wait() for h in range(2)])

    return k


def kernel(tokens, wte, learned_embedding):
    B, S = tokens.shape
    V, D = wte.shape
    N = learned_embedding.shape[1]
    k = _build(B, S, V, D, N)
    out = k(
        tokens.reshape(B * S),
        wte,
        learned_embedding.reshape(B * N, D),
    )
    return out.reshape(B, S, D)


# speedup vs baseline: 6.2302x; 1.0112x over previous
import functools

import jax
import jax.numpy as jnp
from jax import lax
from jax.experimental import pallas as pl
from jax.experimental.pallas import tpu as pltpu
from jax.experimental.pallas import tpu_sc as plsc


@functools.lru_cache(maxsize=None)
def _build(B, S, V, D, N):
    info = plsc.get_sparse_core_info()
    NW = info.num_cores * info.num_subcores

    ROWS = B * S
    assert ROWS % NW == 0
    RPW = ROWS // NW
    C = 32
    NBUF = 3
    assert RPW % C == 0 and C <= 128
    NCH = RPW // C

    assert N <= 32
    offs = [0] + [8 * i for i in range(1, -(-(N - 16) // 8) + 1)] if N > 16 else [0]
    LE_ROWS = offs[-1] + 16
    assert (NBUF * C * D + LE_ROWS * D + RPW) * 4 <= 500 * 1024

    for b in range(B):
        assert (b * S) % RPW == 0 and 1 + N <= C

    mesh = plsc.VectorSubcoreMesh(core_axis_name="c", subcore_axis_name="s")

    @functools.partial(
        pl.kernel,
        out_type=jax.ShapeDtypeStruct((ROWS, D), jnp.float32),
        mesh=mesh,
        scratch_types=[
            pltpu.VMEM((RPW,), jnp.int32),
            [pltpu.VMEM((C, D), jnp.float32) for _ in range(NBUF)],
            pltpu.VMEM((LE_ROWS, D), jnp.float32),
            [pltpu.SemaphoreType.DMA for _ in range(NBUF)],
            [pltpu.SemaphoreType.DMA for _ in range(NBUF)],
            [pltpu.SemaphoreType.DMA for _ in range(len(offs))],
            [pltpu.SemaphoreType.DMA for _ in range(len(offs))],
        ],
    )
    def k(tok_hbm, wte_hbm, le_hbm, out_hbm, idx_v, bufs, le_v,
          gsem, wsem, lsem, ssem):
        wid = lax.axis_index("c") * info.num_subcores + lax.axis_index("s")
        base = wid * RPW

        j16 = lax.iota(jnp.int32, 16)

        def le_gather(b, h):
            gidx = b * N + jnp.minimum(j16 + offs[h], N - 1)
            return pltpu.make_async_copy(
                le_hbm.at[gidx], le_v.at[pl.ds(offs[h], 16)], lsem[h]
            )

        def le_scatter(b, h):
            sidx = b * S + 1 + jnp.minimum(j16 + offs[h], N - 1)
            return pltpu.make_async_copy(
                le_v.at[pl.ds(offs[h], 16)], out_hbm.at[sidx], ssem[h]
            )

        def for_owner(fn):
            for b in range(B):
                owner = (b * S) // RPW

                @pl.when(wid == owner)
                def _():
                    fn(b)

        def splice_issue(b):
            for h in range(len(offs)):
                le_gather(b, h).wait()
            for h in range(len(offs)):
                le_scatter(b, h).start()

        for_owner(lambda b: [le_gather(b, h).start() for h in range(len(offs))])

        pltpu.sync_copy(tok_hbm.at[pl.ds(base, RPW)], idx_v)

        def gather(c):
            return pltpu.async_copy(
                wte_hbm.at[idx_v.at[pl.ds(c * C, C)]], bufs[c % NBUF], gsem[c % NBUF]
            )

        def write(c):
            return pltpu.async_copy(
                bufs[c % NBUF], out_hbm.at[pl.ds(base + c * C, C)], wsem[c % NBUF]
            )

        writes = [None] * NCH
        ghs = [None] * NCH
        spliced = False
        for c in range(NCH):
            if c >= NBUF:
                writes[c - NBUF].wait()
                if not spliced:
                    for_owner(splice_issue)
                    spliced = True
            ghs[c] = gather(c)
            if c >= 1:
                ghs[c - 1].wait()
                writes[c - 1] = write(c - 1)
        ghs[NCH - 1].wait()
        writes[NCH - 1] = write(NCH - 1)
        for c in range(max(0, NCH - NBUF), NCH):
            writes[c].wait()
        if not spliced:
            for_owner(splice_issue)
        for_owner(lambda b: [le_scatter(b, h).wait() for h in range(len(offs))])

    return k


def kernel(tokens, wte, learned_embedding):
    B, S = tokens.shape
    V, D = wte.shape
    N = learned_embedding.shape[1]
    k = _build(B, S, V, D, N)
    out = k(
        tokens.reshape(B * S),
        wte,
        learned_embedding.reshape(B * N, D),
    )
    return out.reshape(B, S, D)
